# Initial kernel scaffold; baseline (speedup 1.0000x reference)
import jax
import jax.numpy as jnp
from jax.experimental import pallas as pl


def kernel(x, edge_index, edge_type, Wb1, Wc1, bc1, b1, Wb2, Wc2, bc2, b2, Wb3, Wc3, bc3, b3):
    raise NotImplementedError("write your pallas kernel here")


# speedup vs baseline: 2.9407x; 2.9407x over previous
import functools

import jax
import jax.numpy as jnp
from jax.experimental import pallas as pl
from jax.experimental.pallas import tpu as pltpu

_N = 10000
_E = 320000
_R = 1000
_G = _N // _R

_f32 = jnp.float32


def _dense1_body(x_ref, wb_ref, wc_ref, bc_ref, p0_ref, p1_ref,
                 bs_ref, w_ref, dis_ref):
    deg = 1.0 + p0_ref[:, 0:1] + p1_ref[:, 0:1]
    dis = jax.lax.rsqrt(deg)
    dis_ref[...] = dis
    b = jnp.dot(x_ref[...], wb_ref[...], preferred_element_type=_f32)
    bs_ref[...] = b * dis
    w_ref[...] = jnp.dot(x_ref[...], wc_ref[...], preferred_element_type=_f32) + bc_ref[...]


def _dense1(x, Wb, Wc, bc, p0, p1):
    fin = x.shape[1]
    return pl.pallas_call(
        _dense1_body,
        grid=(_G,),
        in_specs=[
            pl.BlockSpec((_R, fin), lambda i: (i, 0)),
            pl.BlockSpec((fin, 128), lambda i: (0, 0)),
            pl.BlockSpec((fin, 32), lambda i: (0, 0)),
            pl.BlockSpec((1, 32), lambda i: (0, 0)),
            pl.BlockSpec((_R, 16), lambda i: (i, 0)),
            pl.BlockSpec((_R, 16), lambda i: (i, 0)),
        ],
        out_specs=[
            pl.BlockSpec((_R, 128), lambda i: (i, 0)),
            pl.BlockSpec((_R, 32), lambda i: (i, 0)),
            pl.BlockSpec((_R, 1), lambda i: (i, 0)),
        ],
        out_shape=[
            jax.ShapeDtypeStruct((_N, 128), _f32),
            jax.ShapeDtypeStruct((_N, 32), _f32),
            jax.ShapeDtypeStruct((_N, 1), _f32),
        ],
    )(x, Wb, Wc, bc, p0, p1)


def _dense2_body(x_ref, wb_ref, wc_ref, bc_ref, dis_ref, bs_ref, w_ref):
    b = jnp.dot(x_ref[...], wb_ref[...], preferred_element_type=_f32)
    bs_ref[...] = b * dis_ref[...]
    w_ref[...] = jnp.dot(x_ref[...], wc_ref[...], preferred_element_type=_f32) + bc_ref[...]


def _dense2(x, Wb, Wc, bc, dis):
    fin = x.shape[1]
    return pl.pallas_call(
        _dense2_body,
        grid=(_G,),
        in_specs=[
            pl.BlockSpec((_R, fin), lambda i: (i, 0)),
            pl.BlockSpec((fin, 128), lambda i: (0, 0)),
            pl.BlockSpec((fin, 32), lambda i: (0, 0)),
            pl.BlockSpec((1, 32), lambda i: (0, 0)),
            pl.BlockSpec((_R, 1), lambda i: (i, 0)),
        ],
        out_specs=[
            pl.BlockSpec((_R, 128), lambda i: (i, 0)),
            pl.BlockSpec((_R, 32), lambda i: (i, 0)),
        ],
        out_shape=[
            jax.ShapeDtypeStruct((_N, 128), _f32),
            jax.ShapeDtypeStruct((_N, 32), _f32),
        ],
    )(x, Wb, Wc, bc, dis)


def _combine_math(e0, e1, bs, dis, w, br):
    t = dis * (e0 + e1 + bs)
    parts = []
    for hh in range(8):
        acc = w[:, hh * 4:hh * 4 + 1] * t[:, 0:32]
        for b in range(1, 4):
            acc = acc + w[:, hh * 4 + b:hh * 4 + b + 1] * t[:, b * 32:(b + 1) * 32]
        parts.append(acc)
    return jnp.concatenate(parts, axis=1) + br


def _combine_body(e0_ref, e1_ref, bs_ref, dis_ref, w_ref, br_ref, h_ref):
    out = _combine_math(e0_ref[...], e1_ref[...], bs_ref[...], dis_ref[...],
                        w_ref[...], br_ref[...])
    h_ref[...] = jnp.maximum(out, 0.0)


def _combine_hs_body(e0_ref, e1_ref, bs_ref, dis_ref, w_ref, br_ref,
                     h_ref, hs_ref):
    out = _combine_math(e0_ref[...], e1_ref[...], bs_ref[...], dis_ref[...],
                        w_ref[...], br_ref[...])
    h = jnp.maximum(out, 0.0)
    h_ref[...] = h
    hs_ref[...] = h * dis_ref[...]


def _combine_specs():
    return [
        pl.BlockSpec((_R, 128), lambda i: (i, 0)),
        pl.BlockSpec((_R, 128), lambda i: (i, 0)),
        pl.BlockSpec((_R, 128), lambda i: (i, 0)),
        pl.BlockSpec((_R, 1), lambda i: (i, 0)),
        pl.BlockSpec((_R, 32), lambda i: (i, 0)),
        pl.BlockSpec((1, 256), lambda i: (0, 0)),
    ]


def _combine(e0, e1, bs, dis, w, br):
    return pl.pallas_call(
        _combine_body,
        grid=(_G,),
        in_specs=_combine_specs(),
        out_specs=pl.BlockSpec((_R, 256), lambda i: (i, 0)),
        out_shape=jax.ShapeDtypeStruct((_N, 256), _f32),
    )(e0, e1, bs, dis, w, br)


def _combine_hs(e0, e1, bs, dis, w, br):
    return pl.pallas_call(
        _combine_hs_body,
        grid=(_G,),
        in_specs=_combine_specs(),
        out_specs=[
            pl.BlockSpec((_R, 256), lambda i: (i, 0)),
            pl.BlockSpec((_R, 256), lambda i: (i, 0)),
        ],
        out_shape=[
            jax.ShapeDtypeStruct((_N, 256), _f32),
            jax.ShapeDtypeStruct((_N, 256), _f32),
        ],
    )(e0, e1, bs, dis, w, br)


def _final_body(f0_ref, f1_ref, hs_ref, h_ref, dis_ref, wb_ref, wc_ref,
                bc_ref, b3_ref, o_ref):
    aggH = dis_ref[...] * (jnp.concatenate([f0_ref[...], f1_ref[...]], axis=1)
                           + hs_ref[...])
    aggB = jnp.dot(aggH, wb_ref[...], preferred_element_type=_f32)
    w3 = jnp.dot(h_ref[...], wc_ref[...], preferred_element_type=_f32) + bc_ref[...]
    out = w3[:, 0:1] * aggB[:, 0:128]
    for b in range(1, 4):
        out = out + w3[:, b:b + 1] * aggB[:, b * 128:(b + 1) * 128]
    o_ref[...] = out + b3_ref[...]


def _final(f0, f1, hs, h, dis, Wb3, Wc3, bc3, b3):
    return pl.pallas_call(
        _final_body,
        grid=(_G,),
        in_specs=[
            pl.BlockSpec((_R, 128), lambda i: (i, 0)),
            pl.BlockSpec((_R, 128), lambda i: (i, 0)),
            pl.BlockSpec((_R, 256), lambda i: (i, 0)),
            pl.BlockSpec((_R, 256), lambda i: (i, 0)),
            pl.BlockSpec((_R, 1), lambda i: (i, 0)),
            pl.BlockSpec((256, 512), lambda i: (0, 0)),
            pl.BlockSpec((256, 4), lambda i: (0, 0)),
            pl.BlockSpec((1, 4), lambda i: (0, 0)),
            pl.BlockSpec((1, 128), lambda i: (0, 0)),
        ],
        out_specs=pl.BlockSpec((_R, 128), lambda i: (i, 0)),
        out_shape=jax.ShapeDtypeStruct((_N, 128), _f32),
    )(f0, f1, hs, h, dis, Wb3, Wc3, bc3, b3)


def kernel(x, edge_index, edge_type, Wb1, Wc1, bc1, b1,
           Wb2, Wc2, bc2, b2, Wb3, Wc3, bc3, b3):
    row = edge_index[0]
    col = edge_index[1]

    cnt = jnp.zeros((_N,), _f32).at[col].add(1.0)
    p0 = jnp.broadcast_to(cnt[:, None], (_N, 16))
    p1 = jnp.zeros((_N, 16), _f32)

    bs1, w1, dis = _dense1(x, Wb1, Wc1, bc1.reshape(1, -1), p0, p1)
    e = jnp.zeros((_N, 128), _f32).at[col].add(bs1[row])
    h1 = _combine(e, jnp.zeros_like(e), bs1, dis, w1, b1.reshape(1, -1))

    bs2, w2 = _dense2(h1, Wb2, Wc2, bc2.reshape(1, -1), dis)
    e = jnp.zeros((_N, 128), _f32).at[col].add(bs2[row])
    h2, h2s = _combine_hs(e, jnp.zeros_like(e), bs2, dis, w2, b2.reshape(1, -1))

    f = jnp.zeros((_N, 256), _f32).at[col].add(h2s[row])
    return _final(f[:, :128], f[:, 128:], h2s, h2, dis,
                  Wb3, Wc3, bc3.reshape(1, -1), b3.reshape(1, -1))


# speedup vs baseline: 14.4372x; 4.9095x over previous
import functools

import jax
import jax.numpy as jnp
from jax import lax
from jax.experimental import pallas as pl
from jax.experimental.pallas import tpu as pltpu
from jax.experimental.pallas import tpu_sc as plsc

_N = 10000
_E = 320000
_R = 1000
_G = _N // _R

_f32 = jnp.float32

_NC, _NS = 2, 16
_EROWS, _EW = 2560, 125
_NP = 10240
_RPT = _NP // _NS


def _sc_mesh():
    return plsc.VectorSubcoreMesh(core_axis_name="c", subcore_axis_name="s",
                                  num_cores=_NC, num_subcores=_NS)


def _zero_tile_slice(src, acc, sid):
    for z in range(5):
        pltpu.sync_copy(src, acc.at[pl.ds(sid * _RPT + z * 128, 128)])


def _copy_out(acc, out, cid, sid):
    pltpu.sync_copy(acc.at[pl.ds(sid * _RPT, _RPT)],
                    out.at[pl.ds(cid * _NP + sid * _RPT, _RPT)])


def _deg_body(col_hbm, o_hbm, cibuf, zbuf, obuf, acc):
    cid = lax.axis_index("c")
    sid = lax.axis_index("s")

    def fill(i, c):
        for k in range(8):
            zbuf[i, pl.ds(k * 16, 16)] = jnp.zeros((16,), _f32)
            obuf[i, pl.ds(k * 16, 16)] = jnp.ones((16,), _f32)
        return c

    lax.fori_loop(0, 128, fill, 0)
    _zero_tile_slice(zbuf, acc, sid)
    plsc.subcore_barrier()

    base = cid * (_EROWS // 2) + sid * (_EROWS // 2 // _NS)

    def blk(b, c):
        pltpu.sync_copy(col_hbm.at[pl.ds(base + b * 8, 8)], cibuf)
        for j in range(8):
            pltpu.sync_copy(obuf.at[pl.ds(0, _EW)], acc.at[cibuf.at[j]],
                            add=True)
        return c

    lax.fori_loop(0, _EROWS // 2 // _NS // 8, blk, 0)
    plsc.subcore_barrier()
    _copy_out(acc, o_hbm, cid, sid)


def _sc_deg(col2d):
    k = pl.kernel(
        _deg_body,
        out_type=jax.ShapeDtypeStruct((2 * _NP, 128), _f32),
        mesh=_sc_mesh(),
        scratch_types=[
            pltpu.VMEM((8, _EW), jnp.int32),
            pltpu.VMEM((128, 128), _f32),
            pltpu.VMEM((128, 128), _f32),
            pltpu.VMEM_SHARED((_NP, 128), _f32),
        ],
    )
    return k(col2d)


def _edge_loop(tab, row_hbm, col_hbm, ribuf, cibuf, rbuf, acc, sem,
               base, nblk, col_base=None):
    if col_base is None:
        col_base = base

    def blk(b, c):
        pltpu.sync_copy(row_hbm.at[pl.ds(base + b * 8, 8)], ribuf)
        pltpu.sync_copy(col_hbm.at[pl.ds(col_base + b * 8, 8)], cibuf)
        for j in range(8):
            pltpu.async_copy(tab.at[ribuf.at[j]], rbuf.at[pl.ds(0, _EW)],
                             sem).wait()
            pltpu.sync_copy(rbuf.at[pl.ds(0, _EW)], acc.at[cibuf.at[j]],
                            add=True)
        return c

    lax.fori_loop(0, nblk, blk, 0)


def _zero_rbuf(rbuf):
    def fill(i, c):
        for k in range(8):
            rbuf[i, pl.ds(k * 16, 16)] = jnp.zeros((16,), _f32)
        return c

    lax.fori_loop(0, 128, fill, 0)


def _aggA_body(tab_hbm, row_hbm, col_hbm, o_hbm,
               ribuf, cibuf, rbuf, acc, sem):
    cid = lax.axis_index("c")
    sid = lax.axis_index("s")
    _zero_rbuf(rbuf)
    _zero_tile_slice(rbuf, acc, sid)
    plsc.subcore_barrier()
    base = cid * (_EROWS // 2) + sid * (_EROWS // 2 // _NS)
    _edge_loop(tab_hbm, row_hbm, col_hbm, ribuf, cibuf, rbuf, acc, sem,
               base, _EROWS // 2 // _NS // 8)
    plsc.subcore_barrier()
    _copy_out(acc, o_hbm, cid, sid)


def _aggB_body(tab_hbm, row_hbm, col_hbm, o_hbm,
               ribuf, cibuf, rbuf, acc, sem):
    cid = lax.axis_index("c")
    sid = lax.axis_index("s")
    _zero_rbuf(rbuf)
    _zero_tile_slice(rbuf, acc, sid)
    plsc.subcore_barrier()
    base = cid * _EROWS + sid * (_EROWS // _NS)
    _edge_loop(tab_hbm, row_hbm, col_hbm, ribuf, cibuf, rbuf, acc, sem,
               base, _EROWS // _NS // 8, col_base=sid * (_EROWS // _NS))
    plsc.subcore_barrier()
    _copy_out(acc, o_hbm, cid, sid)


def _agg_scratch():
    return [
        pltpu.VMEM((8, _EW), jnp.int32),
        pltpu.VMEM((8, _EW), jnp.int32),
        pltpu.VMEM((128, 128), _f32),
        pltpu.VMEM_SHARED((_NP, 128), _f32),
        pltpu.SemaphoreType.DMA,
    ]


def _sc_aggA(tab, row2d, col2d):
    k = pl.kernel(
        _aggA_body,
        out_type=jax.ShapeDtypeStruct((2 * _NP, 128), _f32),
        mesh=_sc_mesh(),
        scratch_types=_agg_scratch(),
    )
    return k(tab, row2d, col2d)


def _sc_aggB(tab, rowB, col2d):
    k = pl.kernel(
        _aggB_body,
        out_type=jax.ShapeDtypeStruct((2 * _NP, 128), _f32),
        mesh=_sc_mesh(),
        scratch_types=_agg_scratch(),
    )
    return k(tab, rowB, col2d)


def _dense1_body(x_ref, wb_ref, wc_ref, bc_ref, p0_ref, p1_ref,
                 bs_ref, w_ref, dis_ref):
    deg = 1.0 + p0_ref[:, 0:1] + p1_ref[:, 0:1]
    dis = jax.lax.rsqrt(deg)
    dis_ref[...] = dis
    b = jnp.dot(x_ref[...], wb_ref[...], preferred_element_type=_f32)
    bs_ref[...] = b * dis
    w_ref[...] = jnp.dot(x_ref[...], wc_ref[...], preferred_element_type=_f32) + bc_ref[...]


def _dense1(x, Wb, Wc, bc, p0, p1):
    fin = x.shape[1]
    return pl.pallas_call(
        _dense1_body,
        grid=(_G,),
        in_specs=[
            pl.BlockSpec((_R, fin), lambda i: (i, 0)),
            pl.BlockSpec((fin, 128), lambda i: (0, 0)),
            pl.BlockSpec((fin, 32), lambda i: (0, 0)),
            pl.BlockSpec((1, 32), lambda i: (0, 0)),
            pl.BlockSpec((_R, 128), lambda i: (i, 0)),
            pl.BlockSpec((_R, 128), lambda i: (i, 0)),
        ],
        out_specs=[
            pl.BlockSpec((_R, 128), lambda i: (i, 0)),
            pl.BlockSpec((_R, 32), lambda i: (i, 0)),
            pl.BlockSpec((_R, 1), lambda i: (i, 0)),
        ],
        out_shape=[
            jax.ShapeDtypeStruct((_N, 128), _f32),
            jax.ShapeDtypeStruct((_N, 32), _f32),
            jax.ShapeDtypeStruct((_N, 1), _f32),
        ],
    )(x, Wb, Wc, bc, p0, p1)


def _dense2_body(x_ref, wb_ref, wc_ref, bc_ref, dis_ref, bs_ref, w_ref):
    b = jnp.dot(x_ref[...], wb_ref[...], preferred_element_type=_f32)
    bs_ref[...] = b * dis_ref[...]
    w_ref[...] = jnp.dot(x_ref[...], wc_ref[...], preferred_element_type=_f32) + bc_ref[...]


def _dense2(x, Wb, Wc, bc, dis):
    fin = x.shape[1]
    return pl.pallas_call(
        _dense2_body,
        grid=(_G,),
        in_specs=[
            pl.BlockSpec((_R, fin), lambda i: (i, 0)),
            pl.BlockSpec((fin, 128), lambda i: (0, 0)),
            pl.BlockSpec((fin, 32), lambda i: (0, 0)),
            pl.BlockSpec((1, 32), lambda i: (0, 0)),
            pl.BlockSpec((_R, 1), lambda i: (i, 0)),
        ],
        out_specs=[
            pl.BlockSpec((_R, 128), lambda i: (i, 0)),
            pl.BlockSpec((_R, 32), lambda i: (i, 0)),
        ],
        out_shape=[
            jax.ShapeDtypeStruct((_N, 128), _f32),
            jax.ShapeDtypeStruct((_N, 32), _f32),
        ],
    )(x, Wb, Wc, bc, dis)


def _combine_math(e0, e1, bs, dis, w, br):
    t = dis * (e0 + e1 + bs)
    parts = []
    for hh in range(8):
        acc = w[:, hh * 4:hh * 4 + 1] * t[:, 0:32]
        for b in range(1, 4):
            acc = acc + w[:, hh * 4 + b:hh * 4 + b + 1] * t[:, b * 32:(b + 1) * 32]
        parts.append(acc)
    return jnp.concatenate(parts, axis=1) + br


def _combine_body(e0_ref, e1_ref, bs_ref, dis_ref, w_ref, br_ref, h_ref):
    out = _combine_math(e0_ref[...], e1_ref[...], bs_ref[...], dis_ref[...],
                        w_ref[...], br_ref[...])
    h_ref[...] = jnp.maximum(out, 0.0)


def _combine_hs_body(e0_ref, e1_ref, bs_ref, dis_ref, w_ref, br_ref,
                     h_ref, hs_ref):
    out = _combine_math(e0_ref[...], e1_ref[...], bs_ref[...], dis_ref[...],
                        w_ref[...], br_ref[...])
    h = jnp.maximum(out, 0.0)
    h_ref[...] = h
    hs_ref[...] = h * dis_ref[...]


def _combine_specs():
    return [
        pl.BlockSpec((_R, 128), lambda i: (i, 0)),
        pl.BlockSpec((_R, 128), lambda i: (i, 0)),
        pl.BlockSpec((_R, 128), lambda i: (i, 0)),
        pl.BlockSpec((_R, 1), lambda i: (i, 0)),
        pl.BlockSpec((_R, 32), lambda i: (i, 0)),
        pl.BlockSpec((1, 256), lambda i: (0, 0)),
    ]


def _combine(e0, e1, bs, dis, w, br):
    return pl.pallas_call(
        _combine_body,
        grid=(_G,),
        in_specs=_combine_specs(),
        out_specs=pl.BlockSpec((_R, 256), lambda i: (i, 0)),
        out_shape=jax.ShapeDtypeStruct((_N, 256), _f32),
    )(e0, e1, bs, dis, w, br)


def _combine_hs(e0, e1, bs, dis, w, br):
    return pl.pallas_call(
        _combine_hs_body,
        grid=(_G,),
        in_specs=_combine_specs(),
        out_specs=[
            pl.BlockSpec((_R, 256), lambda i: (i, 0)),
            pl.BlockSpec((_R, 256), lambda i: (i, 0)),
        ],
        out_shape=[
            jax.ShapeDtypeStruct((_N, 256), _f32),
            jax.ShapeDtypeStruct((_N, 256), _f32),
        ],
    )(e0, e1, bs, dis, w, br)


def _final_body(f0_ref, f1_ref, hs_ref, h_ref, dis_ref, wb_ref, wc_ref,
                bc_ref, b3_ref, o_ref):
    aggH = dis_ref[...] * (jnp.concatenate([f0_ref[...], f1_ref[...]], axis=1)
                           + hs_ref[...])
    aggB = jnp.dot(aggH, wb_ref[...], preferred_element_type=_f32)
    w3 = jnp.dot(h_ref[...], wc_ref[...], preferred_element_type=_f32) + bc_ref[...]
    out = w3[:, 0:1] * aggB[:, 0:128]
    for b in range(1, 4):
        out = out + w3[:, b:b + 1] * aggB[:, b * 128:(b + 1) * 128]
    o_ref[...] = out + b3_ref[...]


def _final(f0, f1, hs, h, dis, Wb3, Wc3, bc3, b3):
    return pl.pallas_call(
        _final_body,
        grid=(_G,),
        in_specs=[
            pl.BlockSpec((_R, 128), lambda i: (i, 0)),
            pl.BlockSpec((_R, 128), lambda i: (i, 0)),
            pl.BlockSpec((_R, 256), lambda i: (i, 0)),
            pl.BlockSpec((_R, 256), lambda i: (i, 0)),
            pl.BlockSpec((_R, 1), lambda i: (i, 0)),
            pl.BlockSpec((256, 512), lambda i: (0, 0)),
            pl.BlockSpec((256, 4), lambda i: (0, 0)),
            pl.BlockSpec((1, 4), lambda i: (0, 0)),
            pl.BlockSpec((1, 128), lambda i: (0, 0)),
        ],
        out_specs=pl.BlockSpec((_R, 128), lambda i: (i, 0)),
        out_shape=jax.ShapeDtypeStruct((_N, 128), _f32),
    )(f0, f1, hs, h, dis, Wb3, Wc3, bc3, b3)


def kernel(x, edge_index, edge_type, Wb1, Wc1, bc1, b1,
           Wb2, Wc2, bc2, b2, Wb3, Wc3, bc3, b3):
    row2d = edge_index[0].reshape(_EROWS, _EW)
    col2d = edge_index[1].reshape(_EROWS, _EW)

    d = _sc_deg(col2d)
    p0, p1 = d[:_NP], d[_NP:]

    bs1, w1, dis = _dense1(x, Wb1, Wc1, bc1.reshape(1, -1), p0, p1)
    e = _sc_aggA(bs1, row2d, col2d)
    h1 = _combine(e[:_NP], e[_NP:], bs1, dis, w1, b1.reshape(1, -1))

    bs2, w2 = _dense2(h1, Wb2, Wc2, bc2.reshape(1, -1), dis)
    e = _sc_aggA(bs2, row2d, col2d)
    h2, h2s = _combine_hs(e[:_NP], e[_NP:], bs2, dis, w2, b2.reshape(1, -1))

    tab3 = jnp.concatenate([h2s[:, :128], h2s[:, 128:]], axis=0)
    rowB = jnp.concatenate([row2d, row2d + _N], axis=0)
    f = _sc_aggB(tab3, rowB, col2d)
    return _final(f[:_NP], f[_NP:], h2s, h2, dis,
                  Wb3, Wc3, bc3.reshape(1, -1), b3.reshape(1, -1))


# speedup vs baseline: 17.0145x; 1.1785x over previous
import functools

import jax
import jax.numpy as jnp
from jax import lax
from jax.experimental import pallas as pl
from jax.experimental.pallas import tpu as pltpu
from jax.experimental.pallas import tpu_sc as plsc

_N = 10000
_E = 320000
_R = 1000
_G = _N // _R

_f32 = jnp.float32

_NC, _NS = 2, 16
_EROWS, _EW = 2560, 125
_NP = 10240
_RPT = _NP // _NS


def _sc_mesh():
    return plsc.VectorSubcoreMesh(core_axis_name="c", subcore_axis_name="s",
                                  num_cores=_NC, num_subcores=_NS)


def _zero_tile_slice(src, acc, sid):
    for z in range(5):
        pltpu.sync_copy(src, acc.at[pl.ds(sid * _RPT + z * 128, 128)])


def _copy_out(acc, out, cid, sid):
    pltpu.sync_copy(acc.at[pl.ds(sid * _RPT, _RPT)],
                    out.at[pl.ds(cid * _NP + sid * _RPT, _RPT)])


def _deg_body(col_hbm, o_hbm, cibuf, zbuf, obuf, acc):
    cid = lax.axis_index("c")
    sid = lax.axis_index("s")

    def fill(i, c):
        for k in range(8):
            zbuf[i, pl.ds(k * 16, 16)] = jnp.zeros((16,), _f32)
            obuf[i, pl.ds(k * 16, 16)] = jnp.ones((16,), _f32)
        return c

    lax.fori_loop(0, 128, fill, 0)
    _zero_tile_slice(zbuf, acc, sid)
    plsc.subcore_barrier()

    base = cid * (_EROWS // 2) + sid * (_EROWS // 2 // _NS)

    def blk(b, c):
        pltpu.sync_copy(col_hbm.at[pl.ds(base + b * 8, 8)], cibuf)
        for j in range(8):
            pltpu.sync_copy(obuf.at[pl.ds(0, _EW)], acc.at[cibuf.at[j]],
                            add=True)
        return c

    lax.fori_loop(0, _EROWS // 2 // _NS // 8, blk, 0)
    plsc.subcore_barrier()
    _copy_out(acc, o_hbm, cid, sid)


def _sc_deg(col2d):
    k = pl.kernel(
        _deg_body,
        out_type=jax.ShapeDtypeStruct((2 * _NP, 128), _f32),
        mesh=_sc_mesh(),
        scratch_types=[
            pltpu.VMEM((8, _EW), jnp.int32),
            pltpu.VMEM((128, 128), _f32),
            pltpu.VMEM((128, 128), _f32),
            pltpu.VMEM_SHARED((_NP, 128), _f32),
        ],
    )
    return k(col2d)


def _edge_loop(tab, row_hbm, col_hbm, ribuf, cibuf, rbuf0, rbuf1, acc,
               sem0, sem1, base, nblk, col_base=None):
    if col_base is None:
        col_base = base
    bufs = (rbuf0.at[pl.ds(0, _EW)], rbuf1.at[pl.ds(0, _EW)])
    sems = (sem0, sem1)

    def blk(b, c):
        pltpu.sync_copy(row_hbm.at[pl.ds(base + b * 8, 8)], ribuf)
        pltpu.sync_copy(col_hbm.at[pl.ds(col_base + b * 8, 8)], cibuf)
        handles = [pltpu.async_copy(tab.at[ribuf.at[0]], bufs[0], sems[0])]
        for j in range(8):
            handles[j].wait()
            if j < 7:
                handles.append(pltpu.async_copy(tab.at[ribuf.at[j + 1]],
                                                bufs[(j + 1) % 2],
                                                sems[(j + 1) % 2]))
            pltpu.sync_copy(bufs[j % 2], acc.at[cibuf.at[j]], add=True)
        return c

    lax.fori_loop(0, nblk, blk, 0)


def _zero_rbuf(rbuf):
    def fill(i, c):
        for k in range(8):
            rbuf[i, pl.ds(k * 16, 16)] = jnp.zeros((16,), _f32)
        return c

    lax.fori_loop(0, 128, fill, 0)


def _aggA_body(tab_hbm, row_hbm, col_hbm, o_hbm,
               ribuf, cibuf, rbuf, rbuf1, acc, sem, sem1):
    cid = lax.axis_index("c")
    sid = lax.axis_index("s")
    _zero_rbuf(rbuf)
    _zero_tile_slice(rbuf, acc, sid)
    plsc.subcore_barrier()
    base = cid * (_EROWS // 2) + sid * (_EROWS // 2 // _NS)
    _edge_loop(tab_hbm, row_hbm, col_hbm, ribuf, cibuf, rbuf, rbuf1, acc,
               sem, sem1, base, _EROWS // 2 // _NS // 8)
    plsc.subcore_barrier()
    _copy_out(acc, o_hbm, cid, sid)


def _aggB_body(tab_hbm, row_hbm, col_hbm, o_hbm,
               ribuf, cibuf, rbuf, rbuf1, acc, sem, sem1):
    cid = lax.axis_index("c")
    sid = lax.axis_index("s")
    _zero_rbuf(rbuf)
    _zero_tile_slice(rbuf, acc, sid)
    plsc.subcore_barrier()
    base = cid * _EROWS + sid * (_EROWS // _NS)
    _edge_loop(tab_hbm, row_hbm, col_hbm, ribuf, cibuf, rbuf, rbuf1, acc,
               sem, sem1, base, _EROWS // _NS // 8,
               col_base=sid * (_EROWS // _NS))
    plsc.subcore_barrier()
    _copy_out(acc, o_hbm, cid, sid)


def _agg_scratch():
    return [
        pltpu.VMEM((8, _EW), jnp.int32),
        pltpu.VMEM((8, _EW), jnp.int32),
        pltpu.VMEM((128, 128), _f32),
        pltpu.VMEM((128, 128), _f32),
        pltpu.VMEM_SHARED((_NP, 128), _f32),
        pltpu.SemaphoreType.DMA,
        pltpu.SemaphoreType.DMA,
    ]


def _sc_aggA(tab, row2d, col2d):
    k = pl.kernel(
        _aggA_body,
        out_type=jax.ShapeDtypeStruct((2 * _NP, 128), _f32),
        mesh=_sc_mesh(),
        scratch_types=_agg_scratch(),
    )
    return k(tab, row2d, col2d)


def _sc_aggB(tab, rowB, col2d):
    k = pl.kernel(
        _aggB_body,
        out_type=jax.ShapeDtypeStruct((2 * _NP, 128), _f32),
        mesh=_sc_mesh(),
        scratch_types=_agg_scratch(),
    )
    return k(tab, rowB, col2d)


def _dense1_body(x_ref, wb_ref, wc_ref, bc_ref, p0_ref, p1_ref,
                 bs_ref, w_ref, dis_ref):
    deg = 1.0 + p0_ref[:, 0:1] + p1_ref[:, 0:1]
    dis = jax.lax.rsqrt(deg)
    dis_ref[...] = dis
    b = jnp.dot(x_ref[...], wb_ref[...], preferred_element_type=_f32)
    bs_ref[...] = b * dis
    w_ref[...] = jnp.dot(x_ref[...], wc_ref[...], preferred_element_type=_f32) + bc_ref[...]


def _dense1(x, Wb, Wc, bc, p0, p1):
    fin = x.shape[1]
    return pl.pallas_call(
        _dense1_body,
        grid=(_G,),
        in_specs=[
            pl.BlockSpec((_R, fin), lambda i: (i, 0)),
            pl.BlockSpec((fin, 128), lambda i: (0, 0)),
            pl.BlockSpec((fin, 32), lambda i: (0, 0)),
            pl.BlockSpec((1, 32), lambda i: (0, 0)),
            pl.BlockSpec((_R, 128), lambda i: (i, 0)),
            pl.BlockSpec((_R, 128), lambda i: (i, 0)),
        ],
        out_specs=[
            pl.BlockSpec((_R, 128), lambda i: (i, 0)),
            pl.BlockSpec((_R, 32), lambda i: (i, 0)),
            pl.BlockSpec((_R, 1), lambda i: (i, 0)),
        ],
        out_shape=[
            jax.ShapeDtypeStruct((_N, 128), _f32),
            jax.ShapeDtypeStruct((_N, 32), _f32),
            jax.ShapeDtypeStruct((_N, 1), _f32),
        ],
    )(x, Wb, Wc, bc, p0, p1)


def _dense2_body(x_ref, wb_ref, wc_ref, bc_ref, dis_ref, bs_ref, w_ref):
    b = jnp.dot(x_ref[...], wb_ref[...], preferred_element_type=_f32)
    bs_ref[...] = b * dis_ref[...]
    w_ref[...] = jnp.dot(x_ref[...], wc_ref[...], preferred_element_type=_f32) + bc_ref[...]


def _dense2(x, Wb, Wc, bc, dis):
    fin = x.shape[1]
    return pl.pallas_call(
        _dense2_body,
        grid=(_G,),
        in_specs=[
            pl.BlockSpec((_R, fin), lambda i: (i, 0)),
            pl.BlockSpec((fin, 128), lambda i: (0, 0)),
            pl.BlockSpec((fin, 32), lambda i: (0, 0)),
            pl.BlockSpec((1, 32), lambda i: (0, 0)),
            pl.BlockSpec((_R, 1), lambda i: (i, 0)),
        ],
        out_specs=[
            pl.BlockSpec((_R, 128), lambda i: (i, 0)),
            pl.BlockSpec((_R, 32), lambda i: (i, 0)),
        ],
        out_shape=[
            jax.ShapeDtypeStruct((_N, 128), _f32),
            jax.ShapeDtypeStruct((_N, 32), _f32),
        ],
    )(x, Wb, Wc, bc, dis)


def _combine_math(e0, e1, bs, dis, w, br):
    t = dis * (e0 + e1 + bs)
    parts = []
    for hh in range(8):
        acc = w[:, hh * 4:hh * 4 + 1] * t[:, 0:32]
        for b in range(1, 4):
            acc = acc + w[:, hh * 4 + b:hh * 4 + b + 1] * t[:, b * 32:(b + 1) * 32]
        parts.append(acc)
    return jnp.concatenate(parts, axis=1) + br


def _combine_body(e0_ref, e1_ref, bs_ref, dis_ref, w_ref, br_ref, h_ref):
    out = _combine_math(e0_ref[...], e1_ref[...], bs_ref[...], dis_ref[...],
                        w_ref[...], br_ref[...])
    h_ref[...] = jnp.maximum(out, 0.0)


def _combine_hs_body(e0_ref, e1_ref, bs_ref, dis_ref, w_ref, br_ref,
                     h_ref, hs_ref):
    out = _combine_math(e0_ref[...], e1_ref[...], bs_ref[...], dis_ref[...],
                        w_ref[...], br_ref[...])
    h = jnp.maximum(out, 0.0)
    h_ref[...] = h
    hs_ref[...] = h * dis_ref[...]


def _combine_specs():
    return [
        pl.BlockSpec((_R, 128), lambda i: (i, 0)),
        pl.BlockSpec((_R, 128), lambda i: (i, 0)),
        pl.BlockSpec((_R, 128), lambda i: (i, 0)),
        pl.BlockSpec((_R, 1), lambda i: (i, 0)),
        pl.BlockSpec((_R, 32), lambda i: (i, 0)),
        pl.BlockSpec((1, 256), lambda i: (0, 0)),
    ]


def _combine(e0, e1, bs, dis, w, br):
    return pl.pallas_call(
        _combine_body,
        grid=(_G,),
        in_specs=_combine_specs(),
        out_specs=pl.BlockSpec((_R, 256), lambda i: (i, 0)),
        out_shape=jax.ShapeDtypeStruct((_N, 256), _f32),
    )(e0, e1, bs, dis, w, br)


def _combine_hs(e0, e1, bs, dis, w, br):
    return pl.pallas_call(
        _combine_hs_body,
        grid=(_G,),
        in_specs=_combine_specs(),
        out_specs=[
            pl.BlockSpec((_R, 256), lambda i: (i, 0)),
            pl.BlockSpec((_R, 256), lambda i: (i, 0)),
        ],
        out_shape=[
            jax.ShapeDtypeStruct((_N, 256), _f32),
            jax.ShapeDtypeStruct((_N, 256), _f32),
        ],
    )(e0, e1, bs, dis, w, br)


def _final_body(f0_ref, f1_ref, hs_ref, h_ref, dis_ref, wb_ref, wc_ref,
                bc_ref, b3_ref, o_ref):
    aggH = dis_ref[...] * (jnp.concatenate([f0_ref[...], f1_ref[...]], axis=1)
                           + hs_ref[...])
    aggB = jnp.dot(aggH, wb_ref[...], preferred_element_type=_f32)
    w3 = jnp.dot(h_ref[...], wc_ref[...], preferred_element_type=_f32) + bc_ref[...]
    out = w3[:, 0:1] * aggB[:, 0:128]
    for b in range(1, 4):
        out = out + w3[:, b:b + 1] * aggB[:, b * 128:(b + 1) * 128]
    o_ref[...] = out + b3_ref[...]


def _final(f0, f1, hs, h, dis, Wb3, Wc3, bc3, b3):
    return pl.pallas_call(
        _final_body,
        grid=(_G,),
        in_specs=[
            pl.BlockSpec((_R, 128), lambda i: (i, 0)),
            pl.BlockSpec((_R, 128), lambda i: (i, 0)),
            pl.BlockSpec((_R, 256), lambda i: (i, 0)),
            pl.BlockSpec((_R, 256), lambda i: (i, 0)),
            pl.BlockSpec((_R, 1), lambda i: (i, 0)),
            pl.BlockSpec((256, 512), lambda i: (0, 0)),
            pl.BlockSpec((256, 4), lambda i: (0, 0)),
            pl.BlockSpec((1, 4), lambda i: (0, 0)),
            pl.BlockSpec((1, 128), lambda i: (0, 0)),
        ],
        out_specs=pl.BlockSpec((_R, 128), lambda i: (i, 0)),
        out_shape=jax.ShapeDtypeStruct((_N, 128), _f32),
    )(f0, f1, hs, h, dis, Wb3, Wc3, bc3, b3)


def kernel(x, edge_index, edge_type, Wb1, Wc1, bc1, b1,
           Wb2, Wc2, bc2, b2, Wb3, Wc3, bc3, b3):
    row2d = edge_index[0].reshape(_EROWS, _EW)
    col2d = edge_index[1].reshape(_EROWS, _EW)

    d = _sc_deg(col2d)
    p0, p1 = d[:_NP], d[_NP:]

    bs1, w1, dis = _dense1(x, Wb1, Wc1, bc1.reshape(1, -1), p0, p1)
    e = _sc_aggA(bs1, row2d, col2d)
    h1 = _combine(e[:_NP], e[_NP:], bs1, dis, w1, b1.reshape(1, -1))

    bs2, w2 = _dense2(h1, Wb2, Wc2, bc2.reshape(1, -1), dis)
    e = _sc_aggA(bs2, row2d, col2d)
    h2, h2s = _combine_hs(e[:_NP], e[_NP:], bs2, dis, w2, b2.reshape(1, -1))

    tab3 = jnp.concatenate([h2s[:, :128], h2s[:, 128:]], axis=0)
    rowB = jnp.concatenate([row2d, row2d + _N], axis=0)
    f = _sc_aggB(tab3, rowB, col2d)
    return _final(f[:_NP], f[_NP:], h2s, h2, dis,
                  Wb3, Wc3, bc3.reshape(1, -1), b3.reshape(1, -1))


# speedup vs baseline: 17.3193x; 1.0179x over previous
import functools

import jax
import jax.numpy as jnp
from jax import lax
from jax.experimental import pallas as pl
from jax.experimental.pallas import tpu as pltpu
from jax.experimental.pallas import tpu_sc as plsc

_N = 10000
_E = 320000
_R = 1000
_G = _N // _R

_f32 = jnp.float32

_NC, _NS = 2, 16
_EROWS, _EW = 2560, 125
_NP = 10240
_RPT = _NP // _NS


def _sc_mesh():
    return plsc.VectorSubcoreMesh(core_axis_name="c", subcore_axis_name="s",
                                  num_cores=_NC, num_subcores=_NS)


def _zero_tile_slice(src, acc, sid):
    for z in range(5):
        pltpu.sync_copy(src, acc.at[pl.ds(sid * _RPT + z * 128, 128)])


def _copy_out(acc, out, cid, sid):
    pltpu.sync_copy(acc.at[pl.ds(sid * _RPT, _RPT)],
                    out.at[pl.ds(cid * _NP + sid * _RPT, _RPT)])


def _deg_body(col_hbm, o_hbm, cibuf, zbuf, obuf, acc):
    cid = lax.axis_index("c")
    sid = lax.axis_index("s")

    def fill(i, c):
        for k in range(8):
            zbuf[i, pl.ds(k * 16, 16)] = jnp.zeros((16,), _f32)
            obuf[i, pl.ds(k * 16, 16)] = jnp.ones((16,), _f32)
        return c

    lax.fori_loop(0, 128, fill, 0)
    _zero_tile_slice(zbuf, acc, sid)
    plsc.subcore_barrier()

    base = cid * (_EROWS // 2) + sid * (_EROWS // 2 // _NS)

    def blk(b, c):
        pltpu.sync_copy(col_hbm.at[pl.ds(base + b * 8, 8)], cibuf)
        for j in range(8):
            pltpu.sync_copy(obuf.at[pl.ds(0, _EW)], acc.at[cibuf.at[j]],
                            add=True)
        return c

    lax.fori_loop(0, _EROWS // 2 // _NS // 8, blk, 0)
    plsc.subcore_barrier()
    _copy_out(acc, o_hbm, cid, sid)


def _sc_deg(col2d):
    k = pl.kernel(
        _deg_body,
        out_type=jax.ShapeDtypeStruct((2 * _NP, 128), _f32),
        mesh=_sc_mesh(),
        scratch_types=[
            pltpu.VMEM((8, _EW), jnp.int32),
            pltpu.VMEM((128, 128), _f32),
            pltpu.VMEM((128, 128), _f32),
            pltpu.VMEM_SHARED((_NP, 128), _f32),
        ],
    )
    return k(col2d)


def _edge_loop(tab, row_hbm, col_hbm, ribuf, cibuf, rbuf0, rbuf1, acc,
               sem0, sem1, base, nblk, col_base=None):
    if col_base is None:
        col_base = base
    bufs = (rbuf0.at[pl.ds(0, _EW)], rbuf1.at[pl.ds(0, _EW)])
    sems = (sem0, sem1)

    def blk(b, c):
        pltpu.sync_copy(row_hbm.at[pl.ds(base + b * 8, 8)], ribuf)
        pltpu.sync_copy(col_hbm.at[pl.ds(col_base + b * 8, 8)], cibuf)
        handles = [pltpu.async_copy(tab.at[ribuf.at[0]], bufs[0], sems[0])]
        for j in range(8):
            handles[j].wait()
            if j < 7:
                handles.append(pltpu.async_copy(tab.at[ribuf.at[j + 1]],
                                                bufs[(j + 1) % 2],
                                                sems[(j + 1) % 2]))
            pltpu.sync_copy(bufs[j % 2], acc.at[cibuf.at[j]], add=True)
        return c

    lax.fori_loop(0, nblk, blk, 0)


def _zero_rbuf(rbuf):
    def fill(i, c):
        for k in range(8):
            rbuf[i, pl.ds(k * 16, 16)] = jnp.zeros((16,), _f32)
        return c

    lax.fori_loop(0, 128, fill, 0)


def _aggA_body(tab_hbm, row_hbm, col_hbm, o_hbm,
               ribuf, cibuf, rbuf, rbuf1, acc, sem, sem1):
    cid = lax.axis_index("c")
    sid = lax.axis_index("s")
    _zero_rbuf(rbuf)
    _zero_tile_slice(rbuf, acc, sid)
    plsc.subcore_barrier()
    base = cid * (_EROWS // 2) + sid * (_EROWS // 2 // _NS)
    _edge_loop(tab_hbm, row_hbm, col_hbm, ribuf, cibuf, rbuf, rbuf1, acc,
               sem, sem1, base, _EROWS // 2 // _NS // 8)
    plsc.subcore_barrier()
    _copy_out(acc, o_hbm, cid, sid)


def _aggB_body(tab_hbm, row_hbm, col_hbm, o_hbm,
               ribuf, cibuf, rbuf, rbuf1, acc, sem, sem1):
    cid = lax.axis_index("c")
    sid = lax.axis_index("s")
    _zero_rbuf(rbuf)
    _zero_tile_slice(rbuf, acc, sid)
    plsc.subcore_barrier()
    base = cid * _EROWS + sid * (_EROWS // _NS)
    _edge_loop(tab_hbm, row_hbm, col_hbm, ribuf, cibuf, rbuf, rbuf1, acc,
               sem, sem1, base, _EROWS // _NS // 8,
               col_base=sid * (_EROWS // _NS))
    plsc.subcore_barrier()
    _copy_out(acc, o_hbm, cid, sid)


def _agg_scratch():
    return [
        pltpu.VMEM((8, _EW), jnp.int32),
        pltpu.VMEM((8, _EW), jnp.int32),
        pltpu.VMEM((128, 128), _f32),
        pltpu.VMEM((128, 128), _f32),
        pltpu.VMEM_SHARED((_NP, 128), _f32),
        pltpu.SemaphoreType.DMA,
        pltpu.SemaphoreType.DMA,
    ]


def _sc_aggA(tab, row2d, col2d):
    k = pl.kernel(
        _aggA_body,
        out_type=jax.ShapeDtypeStruct((2 * _NP, 128), _f32),
        mesh=_sc_mesh(),
        scratch_types=_agg_scratch(),
    )
    return k(tab, row2d, col2d)


def _sc_aggB(tab, rowB, col2d):
    k = pl.kernel(
        _aggB_body,
        out_type=jax.ShapeDtypeStruct((2 * _NP, 128), _f32),
        mesh=_sc_mesh(),
        scratch_types=_agg_scratch(),
    )
    return k(tab, rowB, col2d)


def _pre_body(x_ref, wb_ref, wc_ref, bc_ref, braw_ref, w_ref):
    braw_ref[...] = jnp.dot(x_ref[...], wb_ref[...], preferred_element_type=_f32)
    w_ref[...] = jnp.dot(x_ref[...], wc_ref[...], preferred_element_type=_f32) + bc_ref[...]


def _pre(x, Wb, Wc, bc):
    fin = x.shape[1]
    return pl.pallas_call(
        _pre_body,
        grid=(_G,),
        in_specs=[
            pl.BlockSpec((_R, fin), lambda i: (i, 0)),
            pl.BlockSpec((fin, 128), lambda i: (0, 0)),
            pl.BlockSpec((fin, 32), lambda i: (0, 0)),
            pl.BlockSpec((1, 32), lambda i: (0, 0)),
        ],
        out_specs=[
            pl.BlockSpec((_R, 128), lambda i: (i, 0)),
            pl.BlockSpec((_R, 32), lambda i: (i, 0)),
        ],
        out_shape=[
            jax.ShapeDtypeStruct((_N, 128), _f32),
            jax.ShapeDtypeStruct((_N, 32), _f32),
        ],
    )(x, Wb, Wc, bc)


def _scale1_body(braw_ref, p0_ref, p1_ref, bs_ref, dis_ref):
    deg = 1.0 + p0_ref[:, 0:1] + p1_ref[:, 0:1]
    dis = jax.lax.rsqrt(deg)
    dis_ref[...] = dis
    bs_ref[...] = braw_ref[...] * dis


def _scale1(braw, p0, p1):
    return pl.pallas_call(
        _scale1_body,
        grid=(_G,),
        in_specs=[
            pl.BlockSpec((_R, 128), lambda i: (i, 0)),
            pl.BlockSpec((_R, 128), lambda i: (i, 0)),
            pl.BlockSpec((_R, 128), lambda i: (i, 0)),
        ],
        out_specs=[
            pl.BlockSpec((_R, 128), lambda i: (i, 0)),
            pl.BlockSpec((_R, 1), lambda i: (i, 0)),
        ],
        out_shape=[
            jax.ShapeDtypeStruct((_N, 128), _f32),
            jax.ShapeDtypeStruct((_N, 1), _f32),
        ],
    )(braw, p0, p1)


def _combine_math(e0, e1, bs, dis, w, br):
    t = dis * (e0 + e1 + bs)
    parts = []
    for hh in range(8):
        acc = w[:, hh * 4:hh * 4 + 1] * t[:, 0:32]
        for b in range(1, 4):
            acc = acc + w[:, hh * 4 + b:hh * 4 + b + 1] * t[:, b * 32:(b + 1) * 32]
        parts.append(acc)
    return jnp.concatenate(parts, axis=1) + br


def _cd2_body(e0_ref, e1_ref, bs_ref, dis_ref, w_ref, br_ref,
              wb2_ref, wc2_ref, bc2_ref, bs2_ref, w2_ref):
    h1 = jnp.maximum(
        _combine_math(e0_ref[...], e1_ref[...], bs_ref[...], dis_ref[...],
                      w_ref[...], br_ref[...]), 0.0)
    b2 = jnp.dot(h1, wb2_ref[...], preferred_element_type=_f32)
    bs2_ref[...] = b2 * dis_ref[...]
    w2_ref[...] = jnp.dot(h1, wc2_ref[...], preferred_element_type=_f32) + bc2_ref[...]


def _cd2(e0, e1, bs, dis, w, br, Wb2, Wc2, bc2):
    return pl.pallas_call(
        _cd2_body,
        grid=(_G,),
        in_specs=[
            pl.BlockSpec((_R, 128), lambda i: (i, 0)),
            pl.BlockSpec((_R, 128), lambda i: (i, 0)),
            pl.BlockSpec((_R, 128), lambda i: (i, 0)),
            pl.BlockSpec((_R, 1), lambda i: (i, 0)),
            pl.BlockSpec((_R, 32), lambda i: (i, 0)),
            pl.BlockSpec((1, 256), lambda i: (0, 0)),
            pl.BlockSpec((256, 128), lambda i: (0, 0)),
            pl.BlockSpec((256, 32), lambda i: (0, 0)),
            pl.BlockSpec((1, 32), lambda i: (0, 0)),
        ],
        out_specs=[
            pl.BlockSpec((_R, 128), lambda i: (i, 0)),
            pl.BlockSpec((_R, 32), lambda i: (i, 0)),
        ],
        out_shape=[
            jax.ShapeDtypeStruct((_N, 128), _f32),
            jax.ShapeDtypeStruct((_N, 32), _f32),
        ],
    )(e0, e1, bs, dis, w, br, Wb2, Wc2, bc2)


def _cw3_body(e0_ref, e1_ref, bs_ref, dis_ref, w_ref, br_ref,
              wc3_ref, bc3_ref, hs_ref, w3_ref):
    h2 = jnp.maximum(
        _combine_math(e0_ref[...], e1_ref[...], bs_ref[...], dis_ref[...],
                      w_ref[...], br_ref[...]), 0.0)
    hs_ref[...] = h2 * dis_ref[...]
    w3_ref[...] = jnp.dot(h2, wc3_ref[...], preferred_element_type=_f32) + bc3_ref[...]


def _cw3(e0, e1, bs, dis, w, br, Wc3, bc3):
    return pl.pallas_call(
        _cw3_body,
        grid=(_G,),
        in_specs=[
            pl.BlockSpec((_R, 128), lambda i: (i, 0)),
            pl.BlockSpec((_R, 128), lambda i: (i, 0)),
            pl.BlockSpec((_R, 128), lambda i: (i, 0)),
            pl.BlockSpec((_R, 1), lambda i: (i, 0)),
            pl.BlockSpec((_R, 32), lambda i: (i, 0)),
            pl.BlockSpec((1, 256), lambda i: (0, 0)),
            pl.BlockSpec((256, 4), lambda i: (0, 0)),
            pl.BlockSpec((1, 4), lambda i: (0, 0)),
        ],
        out_specs=[
            pl.BlockSpec((_R, 256), lambda i: (i, 0)),
            pl.BlockSpec((_R, 4), lambda i: (i, 0)),
        ],
        out_shape=[
            jax.ShapeDtypeStruct((_N, 256), _f32),
            jax.ShapeDtypeStruct((_N, 4), _f32),
        ],
    )(e0, e1, bs, dis, w, br, Wc3, bc3)


def _final_body(f0_ref, f1_ref, hs_ref, w3_ref, dis_ref, wb_ref,
                b3_ref, o_ref):
    aggH = dis_ref[...] * (jnp.concatenate([f0_ref[...], f1_ref[...]], axis=1)
                           + hs_ref[...])
    aggB = jnp.dot(aggH, wb_ref[...], preferred_element_type=_f32)
    w3 = w3_ref[...]
    out = w3[:, 0:1] * aggB[:, 0:128]
    for b in range(1, 4):
        out = out + w3[:, b:b + 1] * aggB[:, b * 128:(b + 1) * 128]
    o_ref[...] = out + b3_ref[...]


def _final(f0, f1, hs, w3, dis, Wb3, b3):
    return pl.pallas_call(
        _final_body,
        grid=(_G,),
        in_specs=[
            pl.BlockSpec((_R, 128), lambda i: (i, 0)),
            pl.BlockSpec((_R, 128), lambda i: (i, 0)),
            pl.BlockSpec((_R, 256), lambda i: (i, 0)),
            pl.BlockSpec((_R, 4), lambda i: (i, 0)),
            pl.BlockSpec((_R, 1), lambda i: (i, 0)),
            pl.BlockSpec((256, 512), lambda i: (0, 0)),
            pl.BlockSpec((1, 128), lambda i: (0, 0)),
        ],
        out_specs=pl.BlockSpec((_R, 128), lambda i: (i, 0)),
        out_shape=jax.ShapeDtypeStruct((_N, 128), _f32),
    )(f0, f1, hs, w3, dis, Wb3, b3)


def kernel(x, edge_index, edge_type, Wb1, Wc1, bc1, b1,
           Wb2, Wc2, bc2, b2, Wb3, Wc3, bc3, b3):
    row2d = edge_index[0].reshape(_EROWS, _EW)
    col2d = edge_index[1].reshape(_EROWS, _EW)

    d = _sc_deg(col2d)
    braw, w1 = _pre(x, Wb1, Wc1, bc1.reshape(1, -1))
    bs1, dis = _scale1(braw, d[:_NP], d[_NP:])

    e = _sc_aggA(bs1, row2d, col2d)
    bs2, w2 = _cd2(e[:_NP], e[_NP:], bs1, dis, w1, b1.reshape(1, -1),
                   Wb2, Wc2, bc2.reshape(1, -1))

    e = _sc_aggA(bs2, row2d, col2d)
    h2s, w3 = _cw3(e[:_NP], e[_NP:], bs2, dis, w2, b2.reshape(1, -1),
                   Wc3, bc3.reshape(1, -1))

    tab3 = jnp.concatenate([h2s[:, :128], h2s[:, 128:]], axis=0)
    rowB = jnp.concatenate([row2d, row2d + _N], axis=0)
    f = _sc_aggB(tab3, rowB, col2d)
    return _final(f[:_NP], f[_NP:], h2s, w3, dis, Wb3, b3.reshape(1, -1))


# speedup vs baseline: 17.8049x; 1.0280x over previous
import functools

import jax
import jax.numpy as jnp
from jax import lax
from jax.experimental import pallas as pl
from jax.experimental.pallas import tpu as pltpu
from jax.experimental.pallas import tpu_sc as plsc

_N = 10000
_E = 320000
_R = 1024
_G = 10

_f32 = jnp.float32

_NC, _NS = 2, 16
_EROWS, _EW = 2560, 125
_NP = 10240
_RPT = _NP // _NS
_H2 = _NP // _R


def _sc_mesh():
    return plsc.VectorSubcoreMesh(core_axis_name="c", subcore_axis_name="s",
                                  num_cores=_NC, num_subcores=_NS)


def _zero_tile_slice(src, acc, sid):
    for z in range(5):
        pltpu.sync_copy(src, acc.at[pl.ds(sid * _RPT + z * 128, 128)])


def _copy_out(acc, out, cid, sid):
    pltpu.sync_copy(acc.at[pl.ds(sid * _RPT, _RPT)],
                    out.at[pl.ds(cid * _NP + sid * _RPT, _RPT)])


def _deg_body(col_hbm, o_hbm, cibuf, zbuf, obuf, acc):
    cid = lax.axis_index("c")
    sid = lax.axis_index("s")

    def fill(i, c):
        for k in range(8):
            zbuf[i, pl.ds(k * 16, 16)] = jnp.zeros((16,), _f32)
            obuf[i, pl.ds(k * 16, 16)] = jnp.ones((16,), _f32)
        return c

    lax.fori_loop(0, 128, fill, 0)
    _zero_tile_slice(zbuf, acc, sid)
    plsc.subcore_barrier()

    base = cid * (_EROWS // 2) + sid * (_EROWS // 2 // _NS)

    def blk(b, c):
        pltpu.sync_copy(col_hbm.at[pl.ds(base + b * 8, 8)], cibuf)
        for j in range(8):
            pltpu.sync_copy(obuf.at[pl.ds(0, _EW)], acc.at[cibuf.at[j]],
                            add=True)
        return c

    lax.fori_loop(0, _EROWS // 2 // _NS // 8, blk, 0)
    plsc.subcore_barrier()
    _copy_out(acc, o_hbm, cid, sid)


def _sc_deg(col2d):
    k = pl.kernel(
        _deg_body,
        out_type=jax.ShapeDtypeStruct((2 * _NP, 128), _f32),
        mesh=_sc_mesh(),
        scratch_types=[
            pltpu.VMEM((8, _EW), jnp.int32),
            pltpu.VMEM((128, 128), _f32),
            pltpu.VMEM((128, 128), _f32),
            pltpu.VMEM_SHARED((_NP, 128), _f32),
        ],
    )
    return k(col2d)


def _edge_loop(tab, row_hbm, col_hbm, ribuf, cibuf, rbuf0, rbuf1, acc,
               sem0, sem1, base, nblk, col_base=None):
    if col_base is None:
        col_base = base
    bufs = (rbuf0.at[pl.ds(0, _EW)], rbuf1.at[pl.ds(0, _EW)])
    sems = (sem0, sem1)

    def blk(b, c):
        pltpu.sync_copy(row_hbm.at[pl.ds(base + b * 8, 8)], ribuf)
        pltpu.sync_copy(col_hbm.at[pl.ds(col_base + b * 8, 8)], cibuf)
        handles = [pltpu.async_copy(tab.at[ribuf.at[0]], bufs[0], sems[0])]
        for j in range(8):
            handles[j].wait()
            if j < 7:
                handles.append(pltpu.async_copy(tab.at[ribuf.at[j + 1]],
                                                bufs[(j + 1) % 2],
                                                sems[(j + 1) % 2]))
            pltpu.sync_copy(bufs[j % 2], acc.at[cibuf.at[j]], add=True)
        return c

    lax.fori_loop(0, nblk, blk, 0)


def _zero_rbuf(rbuf):
    def fill(i, c):
        for k in range(8):
            rbuf[i, pl.ds(k * 16, 16)] = jnp.zeros((16,), _f32)
        return c

    lax.fori_loop(0, 128, fill, 0)


def _aggA_body(tab_hbm, row_hbm, col_hbm, o_hbm,
               ribuf, cibuf, rbuf, rbuf1, acc, sem, sem1):
    cid = lax.axis_index("c")
    sid = lax.axis_index("s")
    _zero_rbuf(rbuf)
    _zero_tile_slice(rbuf, acc, sid)
    plsc.subcore_barrier()
    base = cid * (_EROWS // 2) + sid * (_EROWS // 2 // _NS)
    _edge_loop(tab_hbm, row_hbm, col_hbm, ribuf, cibuf, rbuf, rbuf1, acc,
               sem, sem1, base, _EROWS // 2 // _NS // 8)
    plsc.subcore_barrier()
    _copy_out(acc, o_hbm, cid, sid)


def _aggB_body(tab_hbm, row_hbm, col_hbm, o_hbm,
               ribuf, cibuf, rbuf, rbuf1, acc, sem, sem1):
    cid = lax.axis_index("c")
    sid = lax.axis_index("s")
    _zero_rbuf(rbuf)
    _zero_tile_slice(rbuf, acc, sid)
    plsc.subcore_barrier()
    base = cid * _EROWS + sid * (_EROWS // _NS)
    _edge_loop(tab_hbm, row_hbm, col_hbm, ribuf, cibuf, rbuf, rbuf1, acc,
               sem, sem1, base, _EROWS // _NS // 8,
               col_base=sid * (_EROWS // _NS))
    plsc.subcore_barrier()
    _copy_out(acc, o_hbm, cid, sid)


def _agg_scratch():
    return [
        pltpu.VMEM((8, _EW), jnp.int32),
        pltpu.VMEM((8, _EW), jnp.int32),
        pltpu.VMEM((128, 128), _f32),
        pltpu.VMEM((128, 128), _f32),
        pltpu.VMEM_SHARED((_NP, 128), _f32),
        pltpu.SemaphoreType.DMA,
        pltpu.SemaphoreType.DMA,
    ]


def _sc_aggA(tab, row2d, col2d):
    k = pl.kernel(
        _aggA_body,
        out_type=jax.ShapeDtypeStruct((2 * _NP, 128), _f32),
        mesh=_sc_mesh(),
        scratch_types=_agg_scratch(),
    )
    return k(tab, row2d, col2d)


def _sc_aggB(tab, rowB, col2d):
    k = pl.kernel(
        _aggB_body,
        out_type=jax.ShapeDtypeStruct((2 * _NP, 128), _f32),
        mesh=_sc_mesh(),
        scratch_types=_agg_scratch(),
    )
    return k(tab, rowB, col2d)


def _pre_body(x_ref, wb_ref, wc_ref, bc_ref, braw_ref, w_ref):
    braw_ref[...] = jnp.dot(x_ref[...], wb_ref[...], preferred_element_type=_f32)
    w_ref[...] = jnp.dot(x_ref[...], wc_ref[...], preferred_element_type=_f32) + bc_ref[...]


def _pre(x, Wb, Wc, bc):
    fin = x.shape[1]
    return pl.pallas_call(
        _pre_body,
        grid=(_G,),
        in_specs=[
            pl.BlockSpec((_R, fin), lambda i: (i, 0)),
            pl.BlockSpec((fin, 128), lambda i: (0, 0)),
            pl.BlockSpec((fin, 32), lambda i: (0, 0)),
            pl.BlockSpec((1, 32), lambda i: (0, 0)),
        ],
        out_specs=[
            pl.BlockSpec((_R, 128), lambda i: (i, 0)),
            pl.BlockSpec((_R, 32), lambda i: (i, 0)),
        ],
        out_shape=[
            jax.ShapeDtypeStruct((_N, 128), _f32),
            jax.ShapeDtypeStruct((_N, 32), _f32),
        ],
    )(x, Wb, Wc, bc)


def _scale1_body(braw_ref, p0_ref, p1_ref, bs_ref, dis_ref):
    deg = 1.0 + p0_ref[:, 0:1] + p1_ref[:, 0:1]
    dis = jax.lax.rsqrt(deg)
    dis_ref[...] = dis
    bs_ref[...] = braw_ref[...] * dis


def _scale1(braw, p0, p1):
    return pl.pallas_call(
        _scale1_body,
        grid=(_G,),
        in_specs=[
            pl.BlockSpec((_R, 128), lambda i: (i, 0)),
            pl.BlockSpec((_R, 128), lambda i: (i, 0)),
            pl.BlockSpec((_R, 128), lambda i: (i + _H2, 0)),
        ],
        out_specs=[
            pl.BlockSpec((_R, 128), lambda i: (i, 0)),
            pl.BlockSpec((_R, 1), lambda i: (i, 0)),
        ],
        out_shape=[
            jax.ShapeDtypeStruct((_N, 128), _f32),
            jax.ShapeDtypeStruct((_N, 1), _f32),
        ],
    )(braw, p0, p1)


def _combine_math(e0, e1, bs, dis, w, br):
    t = dis * (e0 + e1 + bs)
    parts = []
    for hh in range(8):
        acc = w[:, hh * 4:hh * 4 + 1] * t[:, 0:32]
        for b in range(1, 4):
            acc = acc + w[:, hh * 4 + b:hh * 4 + b + 1] * t[:, b * 32:(b + 1) * 32]
        parts.append(acc)
    return jnp.concatenate(parts, axis=1) + br


def _cd2_body(e0_ref, e1_ref, bs_ref, dis_ref, w_ref, br_ref,
              wb2_ref, wc2_ref, bc2_ref, bs2_ref, w2_ref):
    h1 = jnp.maximum(
        _combine_math(e0_ref[...], e1_ref[...], bs_ref[...], dis_ref[...],
                      w_ref[...], br_ref[...]), 0.0)
    b2 = jnp.dot(h1, wb2_ref[...], preferred_element_type=_f32)
    bs2_ref[...] = b2 * dis_ref[...]
    w2_ref[...] = jnp.dot(h1, wc2_ref[...], preferred_element_type=_f32) + bc2_ref[...]


def _cd2(e0, e1, bs, dis, w, br, Wb2, Wc2, bc2):
    return pl.pallas_call(
        _cd2_body,
        grid=(_G,),
        in_specs=[
            pl.BlockSpec((_R, 128), lambda i: (i, 0)),
            pl.BlockSpec((_R, 128), lambda i: (i + _H2, 0)),
            pl.BlockSpec((_R, 128), lambda i: (i, 0)),
            pl.BlockSpec((_R, 1), lambda i: (i, 0)),
            pl.BlockSpec((_R, 32), lambda i: (i, 0)),
            pl.BlockSpec((1, 256), lambda i: (0, 0)),
            pl.BlockSpec((256, 128), lambda i: (0, 0)),
            pl.BlockSpec((256, 32), lambda i: (0, 0)),
            pl.BlockSpec((1, 32), lambda i: (0, 0)),
        ],
        out_specs=[
            pl.BlockSpec((_R, 128), lambda i: (i, 0)),
            pl.BlockSpec((_R, 32), lambda i: (i, 0)),
        ],
        out_shape=[
            jax.ShapeDtypeStruct((_N, 128), _f32),
            jax.ShapeDtypeStruct((_N, 32), _f32),
        ],
    )(e0, e1, bs, dis, w, br, Wb2, Wc2, bc2)


def _cw3_body(e0_ref, e1_ref, bs_ref, dis_ref, w_ref, br_ref,
              wc3_ref, bc3_ref, hs_ref, w3_ref):
    h2 = jnp.maximum(
        _combine_math(e0_ref[...], e1_ref[...], bs_ref[...], dis_ref[...],
                      w_ref[...], br_ref[...]), 0.0)
    hs_ref[...] = h2 * dis_ref[...]
    w3_ref[...] = jnp.dot(h2, wc3_ref[...], preferred_element_type=_f32) + bc3_ref[...]


def _cw3(e0, e1, bs, dis, w, br, Wc3, bc3):
    return pl.pallas_call(
        _cw3_body,
        grid=(_G,),
        in_specs=[
            pl.BlockSpec((_R, 128), lambda i: (i, 0)),
            pl.BlockSpec((_R, 128), lambda i: (i + _H2, 0)),
            pl.BlockSpec((_R, 128), lambda i: (i, 0)),
            pl.BlockSpec((_R, 1), lambda i: (i, 0)),
            pl.BlockSpec((_R, 32), lambda i: (i, 0)),
            pl.BlockSpec((1, 256), lambda i: (0, 0)),
            pl.BlockSpec((256, 4), lambda i: (0, 0)),
            pl.BlockSpec((1, 4), lambda i: (0, 0)),
        ],
        out_specs=[
            pl.BlockSpec((_R, 256), lambda i: (i, 0)),
            pl.BlockSpec((_R, 4), lambda i: (i, 0)),
        ],
        out_shape=[
            jax.ShapeDtypeStruct((_N, 256), _f32),
            jax.ShapeDtypeStruct((_N, 4), _f32),
        ],
    )(e0, e1, bs, dis, w, br, Wc3, bc3)


def _final_body(f0_ref, f1_ref, hs_ref, w3_ref, dis_ref, wb_ref,
                b3_ref, o_ref):
    aggH = dis_ref[...] * (jnp.concatenate([f0_ref[...], f1_ref[...]], axis=1)
                           + hs_ref[...])
    aggB = jnp.dot(aggH, wb_ref[...], preferred_element_type=_f32)
    w3 = w3_ref[...]
    out = w3[:, 0:1] * aggB[:, 0:128]
    for b in range(1, 4):
        out = out + w3[:, b:b + 1] * aggB[:, b * 128:(b + 1) * 128]
    o_ref[...] = out + b3_ref[...]


def _final(f0, f1, hs, w3, dis, Wb3, b3):
    return pl.pallas_call(
        _final_body,
        grid=(_G,),
        in_specs=[
            pl.BlockSpec((_R, 128), lambda i: (i, 0)),
            pl.BlockSpec((_R, 128), lambda i: (i + _H2, 0)),
            pl.BlockSpec((_R, 256), lambda i: (i, 0)),
            pl.BlockSpec((_R, 4), lambda i: (i, 0)),
            pl.BlockSpec((_R, 1), lambda i: (i, 0)),
            pl.BlockSpec((256, 512), lambda i: (0, 0)),
            pl.BlockSpec((1, 128), lambda i: (0, 0)),
        ],
        out_specs=pl.BlockSpec((_R, 128), lambda i: (i, 0)),
        out_shape=jax.ShapeDtypeStruct((_N, 128), _f32),
    )(f0, f1, hs, w3, dis, Wb3, b3)


def kernel(x, edge_index, edge_type, Wb1, Wc1, bc1, b1,
           Wb2, Wc2, bc2, b2, Wb3, Wc3, bc3, b3):
    row2d = edge_index[0].reshape(_EROWS, _EW)
    col2d = edge_index[1].reshape(_EROWS, _EW)

    d = _sc_deg(col2d)
    braw, w1 = _pre(x, Wb1, Wc1, bc1.reshape(1, -1))
    bs1, dis = _scale1(braw, d, d)

    e = _sc_aggA(bs1, row2d, col2d)
    bs2, w2 = _cd2(e, e, bs1, dis, w1, b1.reshape(1, -1),
                   Wb2, Wc2, bc2.reshape(1, -1))

    e = _sc_aggA(bs2, row2d, col2d)
    h2s, w3 = _cw3(e, e, bs2, dis, w2, b2.reshape(1, -1),
                   Wc3, bc3.reshape(1, -1))

    tab3 = jnp.concatenate([h2s[:, :128], h2s[:, 128:]], axis=0)
    rowB = jnp.concatenate([row2d, row2d + _N], axis=0)
    f = _sc_aggB(tab3, rowB, col2d)
    return _final(f, f, h2s, w3, dis, Wb3, b3.reshape(1, -1))
